# Initial kernel scaffold; baseline (speedup 1.0000x reference)
import jax
import jax.numpy as jnp
from jax.experimental import pallas as pl


def kernel(xyz, params):
    raise NotImplementedError("write your pallas kernel here")


# speedup vs baseline: 2.1602x; 2.1602x over previous
import functools

import numpy as np
import jax
import jax.numpy as jnp
from jax import lax
from jax.experimental import pallas as pl
from jax.experimental.pallas import tpu as pltpu

_F32 = jnp.float32
_BN_EPS = 1e-5


def _fold_bn(layers):
    out = []
    for (w, b, g, beta) in layers:
        s = g / jnp.sqrt(1.0 + _BN_EPS)
        out.append(((w * s[:, None]).T, b * s + beta))
    return out


def _fps_body(planes_ref, rows_ref, new_ref, *, npoint, nb, n):
    sub = n // 128
    iota = (lax.broadcasted_iota(jnp.int32, (sub, 128), 0) * 128
            + lax.broadcasted_iota(jnp.int32, (sub, 128), 1))
    X = [planes_ref[b, 0] for b in range(nb)]
    Y = [planes_ref[b, 1] for b in range(nb)]
    Z = [planes_ref[b, 2] for b in range(nb)]

    def body(t, state):
        dists, fars = state
        nds, nfs = [], []
        for b in range(nb):
            row = rows_ref[b, pl.ds(fars[b], 1), :]
            new_ref[b, pl.ds(t, 1), :] = row
            dx = X[b] - row[0, 0]
            dy = Y[b] - row[0, 1]
            dz = Z[b] - row[0, 2]
            d = dx * dx + dy * dy + dz * dz
            nd = jnp.minimum(dists[b], d)
            m = jnp.max(nd)
            nf = jnp.min(jnp.where(nd == m, iota, n))
            nds.append(nd)
            nfs.append(nf)
        return tuple(nds), tuple(nfs)

    init = (tuple(jnp.full((sub, 128), 1e10, _F32) for _ in range(nb)),
            tuple(jnp.int32(0) for _ in range(nb)))
    lax.fori_loop(0, npoint, body, init)


def _fps(rows, planes, npoint):
    nb, n, _ = rows.shape
    return pl.pallas_call(
        functools.partial(_fps_body, npoint=npoint, nb=nb, n=n),
        out_shape=jax.ShapeDtypeStruct((nb, npoint, 3), _F32),
    )(planes.reshape(nb, 3, n // 128, 128), rows)


def _cumsum_lanes(x, n, st):
    s = 1
    while s < n:
        x = x + jnp.concatenate(
            [jnp.zeros((st, s), jnp.int32), x[:, :n - s]], axis=1)
        s *= 2
    return x


def _sqdist(c, pt):
    c2 = jnp.sum(c * c, axis=1, keepdims=True)
    p2 = jnp.sum(pt * pt, axis=0, keepdims=True)
    cp = lax.dot_general(c, pt, (((1,), (0,)), ((), ())),
                         preferred_element_type=_F32)
    return (c2 + p2) - 2.0 * cp


def _select_onehot(sqd, r2, K, st, n):
    mask = sqd <= r2
    rank = _cumsum_lanes(mask.astype(jnp.int32), n, st)
    count = rank[:, n - 1:n]
    jio = lax.broadcasted_iota(jnp.int32, (st, K), 1) + 1
    tgt = jnp.where(jio <= count, jio, 1)
    ranksel = jnp.where(mask, rank, 0)
    oh = (ranksel[:, None, :] == tgt[:, :, None])
    lane = lax.broadcasted_iota(jnp.int32, (st, K, n), 2)
    oh = oh | ((count == 0)[:, :, None] & (lane == n - 1))
    return oh.astype(_F32).reshape(st * K, n)


def _mm(a, w):
    return lax.dot_general(a, w, (((1,), (0,)), ((), ())),
                           preferred_element_type=_F32,
                           precision=lax.Precision.HIGHEST)


def _sa1_body(c_ref, pl_ref, rows_ref, *refs, Ks, r2s, st, n):
    wrefs, orefs = refs[:18], refs[18:]
    c = c_ref[0]
    sqd = _sqdist(c, pl_ref[0])
    rows = rows_ref[0]
    for i in range(3):
        K = Ks[i]
        oh = _select_onehot(sqd, r2s[i], K, st, n)
        g = _mm(oh, rows)
        crep = jnp.broadcast_to(c[:, None, :], (st, K, 3)).reshape(st * K, 3)
        h = g - crep
        for li in range(3):
            w = wrefs[i * 6 + li * 2][...]
            b = wrefs[i * 6 + li * 2 + 1][...]
            h = jnp.maximum(_mm(h, w) + b[None, :], 0.0)
        orefs[i][0] = jnp.max(h.reshape(st, K, -1), axis=1)


def _sa1(new_xyz, planes, rows, branches, Ks, radii, st=8):
    nb, S, _ = new_xyz.shape
    n = rows.shape[1]
    r2s = tuple(np.float32(r * r) for r in radii)
    wargs, wspecs = [], []
    for lyr in branches:
        for (w, b) in lyr:
            wargs += [w, b]
            wspecs += [pl.BlockSpec(w.shape, lambda bi, si: (0, 0)),
                       pl.BlockSpec(b.shape, lambda bi, si: (0,))]
    outs = [jax.ShapeDtypeStruct((nb, S, lyr[-1][0].shape[1]), _F32)
            for lyr in branches]
    ospecs = [pl.BlockSpec((1, st, o.shape[2]), lambda bi, si: (bi, si, 0))
              for o in outs]
    return pl.pallas_call(
        functools.partial(_sa1_body, Ks=Ks, r2s=r2s, st=st, n=n),
        grid=(nb, S // st),
        in_specs=[pl.BlockSpec((1, st, 3), lambda bi, si: (bi, si, 0)),
                  pl.BlockSpec((1, 3, n), lambda bi, si: (bi, 0, 0)),
                  pl.BlockSpec((1, n, 3), lambda bi, si: (bi, 0, 0))]
        + wspecs,
        out_specs=ospecs,
        out_shape=outs,
    )(new_xyz, planes, rows, *wargs)


def _pointfeat_body(p_ref, x_ref, *refs):
    wrefs, orefs = refs[:6], refs[6:]
    p = p_ref[0]
    x = x_ref[0]
    for i in range(3):
        orefs[i][0] = _mm(p, wrefs[2 * i][...]) + _mm(x, wrefs[2 * i + 1][...])


def _pointfeat(l1_pts, l1_xyz, w1ps, w1xs):
    nb, n, cin = l1_pts.shape
    wargs, wspecs = [], []
    for wp, wx in zip(w1ps, w1xs):
        wargs += [wp, wx]
        wspecs += [pl.BlockSpec(wp.shape, lambda bi: (0, 0)),
                   pl.BlockSpec(wx.shape, lambda bi: (0, 0))]
    outs = [jax.ShapeDtypeStruct((nb, n, wp.shape[1]), _F32) for wp in w1ps]
    return pl.pallas_call(
        _pointfeat_body,
        grid=(nb,),
        in_specs=[pl.BlockSpec((1, n, cin), lambda bi: (bi, 0, 0)),
                  pl.BlockSpec((1, n, 3), lambda bi: (bi, 0, 0))] + wspecs,
        out_specs=[pl.BlockSpec((1, n, o.shape[2]), lambda bi: (bi, 0, 0))
                   for o in outs],
        out_shape=outs,
    )(l1_pts, l1_xyz, *wargs)


def _sa2_body(c_ref, pl_ref, *refs, Ks, r2s, st, n):
    arefs = refs[:3]
    wrefs = refs[3:3 + 6 * 3]
    orefs = refs[3 + 18:]
    c = c_ref[0]
    sqd = _sqdist(c, pl_ref[0])
    for i in range(3):
        K = Ks[i]
        oh = _select_onehot(sqd, r2s[i], K, st, n)
        gA = _mm(oh, arefs[i][0])
        w1x = wrefs[6 * i][...]
        b1 = wrefs[6 * i + 1][...]
        tc = b1[None, :] - _mm(c, w1x)
        c1 = tc.shape[1]
        trep = jnp.broadcast_to(tc[:, None, :], (st, K, c1)).reshape(st * K, c1)
        h = jnp.maximum(gA + trep, 0.0)
        for li in (1, 2):
            w = wrefs[6 * i + 2 * li][...]
            b = wrefs[6 * i + 2 * li + 1][...]
            h = jnp.maximum(_mm(h, w) + b[None, :], 0.0)
        orefs[i][0] = jnp.max(h.reshape(st, K, -1), axis=1)


def _sa2(new_xyz, planes, As, branches, w1xs, Ks, radii, st=8):
    nb, S, _ = new_xyz.shape
    n = As[0].shape[1]
    r2s = tuple(np.float32(r * r) for r in radii)
    aspecs = [pl.BlockSpec((1, n, a.shape[2]), lambda bi, si: (bi, 0, 0))
              for a in As]
    wargs, wspecs = [], []
    for i, lyr in enumerate(branches):
        ws = [w1xs[i], lyr[0][1], lyr[1][0], lyr[1][1], lyr[2][0], lyr[2][1]]
        for w in ws:
            wargs.append(w)
            wspecs.append(pl.BlockSpec(
                w.shape, (lambda bi, si: (0, 0)) if w.ndim == 2
                else (lambda bi, si: (0,))))
    outs = [jax.ShapeDtypeStruct((nb, S, lyr[-1][0].shape[1]), _F32)
            for lyr in branches]
    ospecs = [pl.BlockSpec((1, st, o.shape[2]), lambda bi, si: (bi, si, 0))
              for o in outs]
    return pl.pallas_call(
        functools.partial(_sa2_body, Ks=Ks, r2s=r2s, st=st, n=n),
        grid=(nb, S // st),
        in_specs=[pl.BlockSpec((1, st, 3), lambda bi, si: (bi, si, 0)),
                  pl.BlockSpec((1, 3, n), lambda bi, si: (bi, 0, 0))]
        + aspecs + wspecs,
        out_specs=ospecs,
        out_shape=outs,
    )(new_xyz, planes, *As, *wargs)


def _head_body(x_ref, p_ref, *refs, nb, s):
    wrefs, out_ref = refs[:-1], refs[-1]
    feat = jnp.concatenate([x_ref[...], p_ref[...]], axis=1)
    h = feat
    for li in range(3):
        h = jnp.maximum(_mm(h, wrefs[2 * li][...]) + wrefs[2 * li + 1][...][None, :], 0.0)
    hm = jnp.max(h.reshape(nb, s, h.shape[1]), axis=1)
    for li in (3, 4):
        hm = jnp.maximum(_mm(hm, wrefs[2 * li][...]) + wrefs[2 * li + 1][...][None, :], 0.0)
    out_ref[...] = _mm(hm, wrefs[10][...]) + wrefs[11][...][None, :]


def _head(l2_xyz, l2_pts, sa3, fc1, fc2, fc3):
    nb, s, _ = l2_xyz.shape
    layers = list(sa3) + [fc1, fc2, fc3]
    wargs = []
    for (w, b) in layers:
        wargs += [w, b]
    nc = fc3[0].shape[1]
    return pl.pallas_call(
        functools.partial(_head_body, nb=nb, s=s),
        out_shape=jax.ShapeDtypeStruct((nb, nc), _F32),
    )(l2_xyz.reshape(nb * s, 3), l2_pts.reshape(nb * s, -1), *wargs)


def kernel(xyz, params):
    nb, _, n = xyz.shape
    pts = jnp.transpose(xyz, (0, 2, 1))
    sa1 = [_fold_bn(l) for l in params['sa1']]
    sa2 = [_fold_bn(l) for l in params['sa2']]
    sa3 = _fold_bn(params['sa3'])
    fc1 = _fold_bn([params['fc1']])[0]
    fc2 = _fold_bn([params['fc2']])[0]
    fc3 = (params['fc3'][0].T, params['fc3'][1])

    l1_xyz = _fps(pts, xyz, 512)
    l1_parts = _sa1(l1_xyz, xyz, pts, sa1,
                    Ks=(16, 32, 128), radii=(0.1, 0.2, 0.4))
    l1_pts = jnp.concatenate(l1_parts, axis=-1)

    planes2 = jnp.transpose(l1_xyz, (0, 2, 1))
    l2_xyz = _fps(l1_xyz, planes2, 128)
    w1ps = [lyr[0][0][:320, :] for lyr in sa2]
    w1xs = [lyr[0][0][320:, :] for lyr in sa2]
    As = _pointfeat(l1_pts, l1_xyz, w1ps, w1xs)
    l2_parts = _sa2(l2_xyz, planes2, As, sa2, w1xs,
                    Ks=(32, 64, 128), radii=(0.2, 0.4, 0.8))
    l2_pts = jnp.concatenate(l2_parts, axis=-1)

    return _head(l2_xyz, l2_pts, sa3, fc1, fc2, fc3)


# speedup vs baseline: 2.2549x; 1.0438x over previous
import functools

import numpy as np
import jax
import jax.numpy as jnp
from jax import lax
from jax.experimental import pallas as pl
from jax.experimental.pallas import tpu as pltpu

_F32 = jnp.float32
_BN_EPS = 1e-5


def _fold_bn(layers):
    out = []
    for (w, b, g, beta) in layers:
        s = g / jnp.sqrt(1.0 + _BN_EPS)
        out.append(((w * s[:, None]).T, b * s + beta))
    return out


def _fps_body(planes_ref, rows_ref, new_ref, *, npoint, nb, n):
    sub = n // 128
    iota = (lax.broadcasted_iota(jnp.int32, (sub, 128), 0) * 128
            + lax.broadcasted_iota(jnp.int32, (sub, 128), 1))
    X = [planes_ref[b, 0] for b in range(nb)]
    Y = [planes_ref[b, 1] for b in range(nb)]
    Z = [planes_ref[b, 2] for b in range(nb)]

    sub128 = (sub, 128)

    def body(t, state):
        dists, fars = state
        nds, nfs = [], []
        for b in range(nb):
            row = rows_ref[b, pl.ds(fars[b], 1), :]
            new_ref[b, pl.ds(t, 1), :] = row
            dx = X[b] - jnp.broadcast_to(row[0:1, 0:1], sub128)
            dy = Y[b] - jnp.broadcast_to(row[0:1, 1:2], sub128)
            dz = Z[b] - jnp.broadcast_to(row[0:1, 2:3], sub128)
            d = dx * dx + dy * dy + dz * dz
            nd = jnp.minimum(dists[b], d)
            m = jnp.max(nd, keepdims=True)
            masked = jnp.where(nd == jnp.broadcast_to(m, sub128), iota, n)
            nf = jnp.min(masked)
            nds.append(nd)
            nfs.append(nf)
        return tuple(nds), tuple(nfs)

    init = (tuple(jnp.full((sub, 128), 1e10, _F32) for _ in range(nb)),
            tuple(jnp.int32(0) for _ in range(nb)))
    lax.fori_loop(0, npoint, body, init)


def _fps(rows, planes, npoint):
    nb, n, _ = rows.shape
    return pl.pallas_call(
        functools.partial(_fps_body, npoint=npoint, nb=nb, n=n),
        out_shape=jax.ShapeDtypeStruct((nb, npoint, 3), _F32),
    )(planes.reshape(nb, 3, n // 128, 128), rows)


def _cumsum_lanes(x, n, st):
    s = 1
    while s < n:
        x = x + jnp.concatenate(
            [jnp.zeros((st, s), jnp.int32), x[:, :n - s]], axis=1)
        s *= 2
    return x


def _sqdist(c, pt):
    c2 = jnp.sum(c * c, axis=1, keepdims=True)
    p2 = jnp.sum(pt * pt, axis=0, keepdims=True)
    cp = lax.dot_general(c, pt, (((1,), (0,)), ((), ())),
                         preferred_element_type=_F32)
    return (c2 + p2) - 2.0 * cp


def _select_onehot(sqd, r2, K, st, n):
    mask = sqd <= r2
    rank = _cumsum_lanes(mask.astype(jnp.int32), n, st)
    count = rank[:, n - 1:n]
    jio = lax.broadcasted_iota(jnp.int32, (st, K), 1) + 1
    tgt = jnp.where(jio <= count, jio, 1)
    ranksel = jnp.where(mask, rank, 0)
    oh = (ranksel[:, None, :] == tgt[:, :, None])
    lane = lax.broadcasted_iota(jnp.int32, (st, K, n), 2)
    oh = oh | ((count == 0)[:, :, None] & (lane == n - 1))
    return oh.astype(_F32).reshape(st * K, n)


def _mm(a, w):
    return lax.dot_general(a, w, (((1,), (0,)), ((), ())),
                           preferred_element_type=_F32,
                           precision=lax.Precision.HIGHEST)


def _sa1_body(c_ref, pl_ref, rows_ref, *refs, Ks, r2s, st, n):
    wrefs, orefs = refs[:18], refs[18:]
    c = c_ref[0]
    sqd = _sqdist(c, pl_ref[0])
    rows = rows_ref[0]
    for i in range(3):
        K = Ks[i]
        oh = _select_onehot(sqd, r2s[i], K, st, n)
        g = _mm(oh, rows)
        crep = jnp.broadcast_to(c[:, None, :], (st, K, 3)).reshape(st * K, 3)
        h = g - crep
        for li in range(3):
            w = wrefs[i * 6 + li * 2][...]
            b = wrefs[i * 6 + li * 2 + 1][...]
            h = jnp.maximum(_mm(h, w) + b[None, :], 0.0)
        orefs[i][0] = jnp.max(h.reshape(st, K, -1), axis=1)


def _sa1(new_xyz, planes, rows, branches, Ks, radii, st=16):
    nb, S, _ = new_xyz.shape
    n = rows.shape[1]
    r2s = tuple(np.float32(r * r) for r in radii)
    wargs, wspecs = [], []
    for lyr in branches:
        for (w, b) in lyr:
            wargs += [w, b]
            wspecs += [pl.BlockSpec(w.shape, lambda bi, si: (0, 0)),
                       pl.BlockSpec(b.shape, lambda bi, si: (0,))]
    outs = [jax.ShapeDtypeStruct((nb, S, lyr[-1][0].shape[1]), _F32)
            for lyr in branches]
    ospecs = [pl.BlockSpec((1, st, o.shape[2]), lambda bi, si: (bi, si, 0))
              for o in outs]
    return pl.pallas_call(
        functools.partial(_sa1_body, Ks=Ks, r2s=r2s, st=st, n=n),
        grid=(nb, S // st),
        in_specs=[pl.BlockSpec((1, st, 3), lambda bi, si: (bi, si, 0)),
                  pl.BlockSpec((1, 3, n), lambda bi, si: (bi, 0, 0)),
                  pl.BlockSpec((1, n, 3), lambda bi, si: (bi, 0, 0))]
        + wspecs,
        out_specs=ospecs,
        out_shape=outs,
    )(new_xyz, planes, rows, *wargs)


def _pointfeat_body(p_ref, x_ref, *refs):
    wrefs, orefs = refs[:6], refs[6:]
    p = p_ref[0]
    x = x_ref[0]
    for i in range(3):
        orefs[i][0] = _mm(p, wrefs[2 * i][...]) + _mm(x, wrefs[2 * i + 1][...])


def _pointfeat(l1_pts, l1_xyz, w1ps, w1xs):
    nb, n, cin = l1_pts.shape
    wargs, wspecs = [], []
    for wp, wx in zip(w1ps, w1xs):
        wargs += [wp, wx]
        wspecs += [pl.BlockSpec(wp.shape, lambda bi: (0, 0)),
                   pl.BlockSpec(wx.shape, lambda bi: (0, 0))]
    outs = [jax.ShapeDtypeStruct((nb, n, wp.shape[1]), _F32) for wp in w1ps]
    return pl.pallas_call(
        _pointfeat_body,
        grid=(nb,),
        in_specs=[pl.BlockSpec((1, n, cin), lambda bi: (bi, 0, 0)),
                  pl.BlockSpec((1, n, 3), lambda bi: (bi, 0, 0))] + wspecs,
        out_specs=[pl.BlockSpec((1, n, o.shape[2]), lambda bi: (bi, 0, 0))
                   for o in outs],
        out_shape=outs,
    )(l1_pts, l1_xyz, *wargs)


def _sa2_body(c_ref, pl_ref, *refs, Ks, r2s, st, n):
    arefs = refs[:3]
    wrefs = refs[3:3 + 6 * 3]
    orefs = refs[3 + 18:]
    c = c_ref[0]
    sqd = _sqdist(c, pl_ref[0])
    for i in range(3):
        K = Ks[i]
        oh = _select_onehot(sqd, r2s[i], K, st, n)
        gA = _mm(oh, arefs[i][0])
        w1x = wrefs[6 * i][...]
        b1 = wrefs[6 * i + 1][...]
        tc = b1[None, :] - _mm(c, w1x)
        c1 = tc.shape[1]
        trep = jnp.broadcast_to(tc[:, None, :], (st, K, c1)).reshape(st * K, c1)
        h = jnp.maximum(gA + trep, 0.0)
        for li in (1, 2):
            w = wrefs[6 * i + 2 * li][...]
            b = wrefs[6 * i + 2 * li + 1][...]
            h = jnp.maximum(_mm(h, w) + b[None, :], 0.0)
        orefs[i][0] = jnp.max(h.reshape(st, K, -1), axis=1)


def _sa2(new_xyz, planes, As, branches, w1xs, Ks, radii, st=16):
    nb, S, _ = new_xyz.shape
    n = As[0].shape[1]
    r2s = tuple(np.float32(r * r) for r in radii)
    aspecs = [pl.BlockSpec((1, n, a.shape[2]), lambda bi, si: (bi, 0, 0))
              for a in As]
    wargs, wspecs = [], []
    for i, lyr in enumerate(branches):
        ws = [w1xs[i], lyr[0][1], lyr[1][0], lyr[1][1], lyr[2][0], lyr[2][1]]
        for w in ws:
            wargs.append(w)
            wspecs.append(pl.BlockSpec(
                w.shape, (lambda bi, si: (0, 0)) if w.ndim == 2
                else (lambda bi, si: (0,))))
    outs = [jax.ShapeDtypeStruct((nb, S, lyr[-1][0].shape[1]), _F32)
            for lyr in branches]
    ospecs = [pl.BlockSpec((1, st, o.shape[2]), lambda bi, si: (bi, si, 0))
              for o in outs]
    return pl.pallas_call(
        functools.partial(_sa2_body, Ks=Ks, r2s=r2s, st=st, n=n),
        grid=(nb, S // st),
        in_specs=[pl.BlockSpec((1, st, 3), lambda bi, si: (bi, si, 0)),
                  pl.BlockSpec((1, 3, n), lambda bi, si: (bi, 0, 0))]
        + aspecs + wspecs,
        out_specs=ospecs,
        out_shape=outs,
    )(new_xyz, planes, *As, *wargs)


def _head_body(x_ref, p_ref, *refs, nb, s):
    wrefs, out_ref = refs[:-1], refs[-1]
    feat = jnp.concatenate([x_ref[...], p_ref[...]], axis=1)
    h = feat
    for li in range(3):
        h = jnp.maximum(_mm(h, wrefs[2 * li][...]) + wrefs[2 * li + 1][...][None, :], 0.0)
    hm = jnp.max(h.reshape(nb, s, h.shape[1]), axis=1)
    for li in (3, 4):
        hm = jnp.maximum(_mm(hm, wrefs[2 * li][...]) + wrefs[2 * li + 1][...][None, :], 0.0)
    out_ref[...] = _mm(hm, wrefs[10][...]) + wrefs[11][...][None, :]


def _head(l2_xyz, l2_pts, sa3, fc1, fc2, fc3):
    nb, s, _ = l2_xyz.shape
    layers = list(sa3) + [fc1, fc2, fc3]
    wargs = []
    for (w, b) in layers:
        wargs += [w, b]
    nc = fc3[0].shape[1]
    return pl.pallas_call(
        functools.partial(_head_body, nb=nb, s=s),
        out_shape=jax.ShapeDtypeStruct((nb, nc), _F32),
    )(l2_xyz.reshape(nb * s, 3), l2_pts.reshape(nb * s, -1), *wargs)


def kernel(xyz, params):
    nb, _, n = xyz.shape
    pts = jnp.transpose(xyz, (0, 2, 1))
    sa1 = [_fold_bn(l) for l in params['sa1']]
    sa2 = [_fold_bn(l) for l in params['sa2']]
    sa3 = _fold_bn(params['sa3'])
    fc1 = _fold_bn([params['fc1']])[0]
    fc2 = _fold_bn([params['fc2']])[0]
    fc3 = (params['fc3'][0].T, params['fc3'][1])

    l1_xyz = _fps(pts, xyz, 512)
    l1_parts = _sa1(l1_xyz, xyz, pts, sa1,
                    Ks=(16, 32, 128), radii=(0.1, 0.2, 0.4))
    l1_pts = jnp.concatenate(l1_parts, axis=-1)

    planes2 = jnp.transpose(l1_xyz, (0, 2, 1))
    l2_xyz = _fps(l1_xyz, planes2, 128)
    w1ps = [lyr[0][0][:320, :] for lyr in sa2]
    w1xs = [lyr[0][0][320:, :] for lyr in sa2]
    As = _pointfeat(l1_pts, l1_xyz, w1ps, w1xs)
    l2_parts = _sa2(l2_xyz, planes2, As, sa2, w1xs,
                    Ks=(32, 64, 128), radii=(0.2, 0.4, 0.8))
    l2_pts = jnp.concatenate(l2_parts, axis=-1)

    return _head(l2_xyz, l2_pts, sa3, fc1, fc2, fc3)
